# Initial kernel scaffold; baseline (speedup 1.0000x reference)
import jax
import jax.numpy as jnp
from jax.experimental import pallas as pl


def kernel(feats, points, params):
    raise NotImplementedError("write your pallas kernel here")


# speedup vs baseline: 15.8692x; 15.8692x over previous
import functools

import jax
import jax.numpy as jnp
from jax import lax
from jax.experimental import pallas as pl
from jax.experimental.pallas import tpu as pltpu
from jax.experimental.pallas import tpu_sc as plsc

_K = 16
_EPS = 1e-5
def _dot(a, b):
    return jnp.dot(a, b, preferred_element_type=jnp.float32)


def _gelu(x):
    return 0.5 * x * (1.0 + lax.erf(x * 0.7071067811865476))


def _knn_body(n, nbq, pts_ref, ptsq_ref, idx_ref):
    b = pl.program_id(0)
    p_all = pts_ref[0]
    q3 = ptsq_ref[0]
    sq_all = jnp.sum(p_all * p_all, axis=1, keepdims=True)
    sq_q = jnp.sum(q3 * q3, axis=0, keepdims=True)
    g = lax.dot_general(p_all, q3, (((1,), (0,)), ((), ())),
                        preferred_element_type=jnp.float32)
    d2 = sq_all + sq_q - 2.0 * g
    iota = lax.broadcasted_iota(jnp.int32, d2.shape, 0)
    rows = []
    for _ in range(_K):
        m = jnp.min(d2, axis=0, keepdims=True)
        am = jnp.min(jnp.where(d2 == m, iota, jnp.int32(n)),
                     axis=0, keepdims=True)
        rows.append(am)
        d2 = jnp.where(iota == am, jnp.float32(jnp.inf), d2)
    idx_ref[...] = jnp.concatenate(rows, axis=0) + b * n


def _knn(points, rq=256):
    bb, n, _ = points.shape
    nbq = n // rq
    ptst = points.transpose(0, 2, 1)
    return pl.pallas_call(
        functools.partial(_knn_body, n, nbq),
        grid=(bb, nbq),
        in_specs=[
            pl.BlockSpec((1, n, 3), lambda b, i: (b, 0, 0)),
            pl.BlockSpec((1, 3, rq), lambda b, i: (b, 0, i)),
        ],
        out_specs=pl.BlockSpec((_K, rq), lambda b, i: (0, b * nbq + i)),
        out_shape=jax.ShapeDtypeStruct((_K, bb * n), jnp.int32),
    )(points, ptst)


def _gather(tf, tp, idx3):
    nw, nch, lane = idx3.shape
    kt = nw * nch * lane
    per_w = nch * lane
    mesh = plsc.VectorSubcoreMesh(core_axis_name="c", subcore_axis_name="s")
    info = plsc.get_sparse_core_info()
    nc = info.num_cores

    @functools.partial(
        pl.kernel,
        mesh=mesh,
        compiler_params=pltpu.CompilerParams(use_tc_tiling_on_sc=False),
        out_type=[
            jax.ShapeDtypeStruct((kt, tf.shape[1]), jnp.float32),
            jax.ShapeDtypeStruct((kt, tp.shape[1]), jnp.float32),
        ],
        scratch_types=[
            pltpu.VMEM((nch, lane), jnp.int32),
            pltpu.VMEM((lane, tf.shape[1]), jnp.float32),
            pltpu.VMEM((lane, tp.shape[1]), jnp.float32),
            pltpu.SemaphoreType.DMA,
            pltpu.SemaphoreType.DMA,
        ],
    )
    def run(tf_hbm, tp_hbm, idx_hbm, gf_hbm, gp_hbm, idx_v, fbuf, pbuf,
            sem_f, sem_p):
        wid = lax.axis_index("s") * nc + lax.axis_index("c")
        pltpu.sync_copy(idx_hbm.at[wid], idx_v)

        def body(j, carry):
            base = wid * per_w + j * lane
            cf = pltpu.async_copy(tf_hbm.at[idx_v.at[j]], fbuf, sem_f)
            cp = pltpu.async_copy(tp_hbm.at[idx_v.at[j]], pbuf, sem_p)
            cf.wait()
            cp.wait()
            pltpu.sync_copy(fbuf, gf_hbm.at[pl.ds(base, lane)])
            pltpu.sync_copy(pbuf, gp_hbm.at[pl.ds(base, lane)])
            return carry

        lax.fori_loop(0, nch, body, 0)

    return run(tf, tp, idx3)


def _attn_body(r4, cnt, feats_ref, pts_ref, gf_ref, gp_ref,
               wq_ref, wk_ref, wv_ref, wg1_ref, wg2_ref, wd1_ref, wd2_ref,
               bq_ref, bk_ref, bv_ref, bg1_ref, bg2_ref, bd1_ref, bd2_ref,
               w1_ref, b1_ref, w2_ref, b2_ref, w3_ref, b3_ref,
               out_ref, acc, stats):
    p = pl.program_id(0)
    i = pl.program_id(1)
    rk = _K * r4

    def t_k(x):
        return jnp.concatenate([x] * _K, axis=0)

    def s_k(x):
        a = x[0:r4]
        for k in range(1, _K):
            a = a + x[k * r4:(k + 1) * r4]
        return a

    def m_k(x):
        a = x[0:r4]
        for k in range(1, _K):
            a = jnp.maximum(a, x[k * r4:(k + 1) * r4])
        return a

    @pl.when(jnp.logical_and(p == 0, i == 0))
    def _init():
        acc[...] = jnp.zeros_like(acc)

    def _fold4(v, l):
        return v[:, 0:l] + v[:, l:2 * l] + v[:, 2 * l:3 * l] + v[:, 3 * l:4 * l]

    def _finalize(src, dst, l):
        s = _fold4(acc[src:src + 1, 0:4 * l], l) * (1.0 / cnt)
        ss = _fold4(acc[src + 1:src + 2, 0:4 * l], l) * (1.0 / cnt)
        r = lax.rsqrt(ss - s * s + _EPS)
        stats[dst:dst + 1, 0:4 * l] = jnp.concatenate([s] * 4, axis=1)
        stats[dst + 1:dst + 2, 0:4 * l] = jnp.concatenate([r] * 4, axis=1)

    @pl.when(jnp.logical_and(p == 1, i == 0))
    def _fin1():
        _finalize(0, 0, 16)

    @pl.when(jnp.logical_and(p == 2, i == 0))
    def _fin2():
        _finalize(2, 2, 32)

    @pl.when(jnp.logical_and(p == 3, i == 0))
    def _fin3():
        _finalize(4, 4, 32)

    gp2 = gp_ref[...].reshape(rk, 64)
    dp = t_k(pts_ref[...]) - gp2
    pos_raw = _dot(dp, wd1_ref[...]) + bd1_ref[...]

    @pl.when(p == 0)
    def _acc1():
        acc[0:1, 0:64] += jnp.sum(pos_raw, axis=0, keepdims=True)
        acc[1:2, 0:64] += jnp.sum(pos_raw * pos_raw, axis=0, keepdims=True)

    @pl.when(p >= 1)
    def _main():
        pos_d = _gelu((pos_raw - stats[0:1, 0:64]) * stats[1:2, 0:64]
                      * w1_ref[...] + b1_ref[...])
        pos32 = _dot(pos_d, wd2_ref[...]) + bd2_ref[...]
        q = _dot(feats_ref[...], wq_ref[...]) + bq_ref[...]
        gf2 = gf_ref[...].reshape(rk, 128)
        kk = _dot(gf2, wk_ref[...]) + bk_ref[...]
        gamma0 = t_k(q) - kk + pos32

        @pl.when(p == 1)
        def _acc2():
            acc[2:3, :] += jnp.sum(gamma0, axis=0, keepdims=True)
            acc[3:4, :] += jnp.sum(gamma0 * gamma0, axis=0, keepdims=True)

        @pl.when(p >= 2)
        def _main2():
            g1 = _gelu((gamma0 - stats[2:3, :]) * stats[3:4, :]
                       * w2_ref[...] + b2_ref[...])
            g2 = _dot(g1, wg1_ref[...]) + bg1_ref[...]

            @pl.when(p == 2)
            def _acc3():
                acc[4:5, :] += jnp.sum(g2, axis=0, keepdims=True)
                acc[5:6, :] += jnp.sum(g2 * g2, axis=0, keepdims=True)

            @pl.when(p == 3)
            def _final():
                g3 = _gelu((g2 - stats[4:5, :]) * stats[5:6, :]
                           * w3_ref[...] + b3_ref[...])
                g4 = _dot(g3, wg2_ref[...]) + bg2_ref[...]
                mx = m_k(g4)
                e = jnp.exp(g4 - t_k(mx))
                rho = e / t_k(s_k(e))
                val = _dot(gf2, wv_ref[...]) + bv_ref[...] + pos32
                out_ref[...] = s_k(rho * val)


def _attn(featsp, ptsp, gfp, gpp, wmats, wvecs, r4=128):
    bn4 = featsp.shape[0]
    nblk = bn4 // r4
    cnt = float(bn4 * 4 * _K)
    full = lambda s: pl.BlockSpec(s, lambda p, i: (0,) * len(s))
    in_specs = (
        [pl.BlockSpec((r4, 128), lambda p, i: (i, 0)),
         pl.BlockSpec((r4, 64), lambda p, i: (i, 0)),
         pl.BlockSpec((_K, r4, 128), lambda p, i: (0, i, 0)),
         pl.BlockSpec((_K, r4, 64), lambda p, i: (0, i, 0))]
        + [full(w.shape) for w in wmats]
        + [full(v.shape) for v in wvecs]
    )
    return pl.pallas_call(
        functools.partial(_attn_body, r4, cnt),
        grid=(4, nblk),
        in_specs=in_specs,
        out_specs=pl.BlockSpec((r4, 128), lambda p, i: (i, 0)),
        out_shape=jax.ShapeDtypeStruct((bn4, 128), jnp.float32),
        scratch_shapes=[pltpu.VMEM((8, 128), jnp.float32),
                        pltpu.VMEM((8, 128), jnp.float32)],
    )(featsp, ptsp, gfp, gpp, *wmats, *wvecs)


def kernel(feats, points, params):
    bb, n, df = feats.shape
    bn = bb * n
    f32 = jnp.float32

    idx_t = _knn(points)

    tf = feats.reshape(bn, df)
    tp = jnp.pad(points.reshape(bn, 3), ((0, 0), (0, 13)))
    idx3 = idx_t.reshape(32, (_K * bn) // (32 * 128), 128)
    gf, gp = _gather(tf, tp, idx3)

    eye4 = jnp.eye(4, dtype=f32)
    kron4 = lambda w: jnp.kron(eye4, w)
    wd1_16 = jnp.zeros((16, 16), f32).at[0:3, 0:3].set(params['Wd1'])
    wd2_16 = jnp.zeros((16, df), f32).at[0:3, :].set(params['Wd2'])
    wmats = [kron4(params['Wq']), kron4(params['Wk']), kron4(params['Wv']),
             kron4(params['Wg1']), kron4(params['Wg2']),
             kron4(wd1_16), kron4(wd2_16)]
    t4 = lambda v: jnp.tile(v, 4).reshape(1, -1)
    pad16 = lambda v: jnp.pad(v, (0, 13))
    wvecs = [t4(params['bq']), t4(params['bk']), t4(params['bv']),
             t4(params['bg1']), t4(params['bg2']),
             t4(pad16(params['bd1'])), t4(params['bd2']),
             t4(pad16(params['bn_delta_w'])), t4(pad16(params['bn_delta_b'])),
             t4(params['bn_g1_w']), t4(params['bn_g1_b']),
             t4(params['bn_g2_w']), t4(params['bn_g2_b'])]

    featsp = feats.reshape(bn // 4, 128)
    ptsp = tp.reshape(bn // 4, 64)
    gfp = gf.reshape(_K, bn // 4, 128)
    gpp = gp.reshape(_K, bn // 4, 64)
    outp = _attn(featsp, ptsp, gfp, gpp, wmats, wvecs)
    return outp.reshape(bb, n, df)


# speedup vs baseline: 18.1357x; 1.1428x over previous
import functools

import jax
import jax.numpy as jnp
from jax import lax
from jax.experimental import pallas as pl
from jax.experimental.pallas import tpu as pltpu
from jax.experimental.pallas import tpu_sc as plsc

_K = 16
_EPS = 1e-5
def _dot(a, b):
    return jnp.dot(a, b, preferred_element_type=jnp.float32)


def _gelu(x):
    return 0.5 * x * (1.0 + lax.erf(x * 0.7071067811865476))


def _knn_body(n, nbq, pts_ref, ptsq_ref, idx_ref):
    b = pl.program_id(0)
    p_all = pts_ref[0]
    q3 = ptsq_ref[0]
    sq_all = jnp.sum(p_all * p_all, axis=1, keepdims=True)
    sq_q = jnp.sum(q3 * q3, axis=0, keepdims=True)
    g = lax.dot_general(p_all * -2.0, q3, (((1,), (0,)), ((), ())),
                        preferred_element_type=jnp.float32)
    d2 = (sq_all + sq_q) + g
    fiota = lax.broadcasted_iota(jnp.int32, d2.shape, 0).astype(jnp.float32)
    big = jnp.float32(n)
    rows = []
    for _ in range(_K):
        m = jnp.min(d2, axis=0, keepdims=True)
        am = jnp.min(jnp.where(d2 == m, fiota, big), axis=0, keepdims=True)
        rows.append(am)
        d2 = jnp.where(fiota == am, jnp.float32(jnp.inf), d2)
    idx_ref[...] = jnp.concatenate(rows, axis=0).astype(jnp.int32) + b * n


def _knn(points, rq=512):
    bb, n, _ = points.shape
    nbq = n // rq
    ptst = points.transpose(0, 2, 1)
    return pl.pallas_call(
        functools.partial(_knn_body, n, nbq),
        grid=(bb, nbq),
        in_specs=[
            pl.BlockSpec((1, n, 3), lambda b, i: (b, 0, 0)),
            pl.BlockSpec((1, 3, rq), lambda b, i: (b, 0, i)),
        ],
        out_specs=pl.BlockSpec((_K, rq), lambda b, i: (0, b * nbq + i)),
        out_shape=jax.ShapeDtypeStruct((_K, bb * n), jnp.int32),
    )(points, ptst)


def _gather(tf, tp, idx3):
    nw, nch, lane = idx3.shape
    kt = nw * nch * lane
    per_w = nch * lane
    mesh = plsc.VectorSubcoreMesh(core_axis_name="c", subcore_axis_name="s")
    info = plsc.get_sparse_core_info()
    nc = info.num_cores

    nb = 8
    @functools.partial(
        pl.kernel,
        mesh=mesh,
        compiler_params=pltpu.CompilerParams(use_tc_tiling_on_sc=False),
        out_type=[
            jax.ShapeDtypeStruct((kt, tf.shape[1]), jnp.float32),
            jax.ShapeDtypeStruct((kt, tp.shape[1]), jnp.float32),
        ],
        scratch_types=[
            pltpu.VMEM((nch, lane), jnp.int32),
            pltpu.VMEM((nb, lane, tf.shape[1]), jnp.float32),
            pltpu.VMEM((nb, lane, tp.shape[1]), jnp.float32),
            pltpu.SemaphoreType.DMA,
            pltpu.SemaphoreType.DMA,
        ],
    )
    def run(tf_hbm, tp_hbm, idx_hbm, gf_hbm, gp_hbm, idx_v, fbuf, pbuf,
            sem_g, sem_o):
        wid = lax.axis_index("s") * nc + lax.axis_index("c")
        pltpu.sync_copy(idx_hbm.at[wid], idx_v)

        def body(j8, carry):
            gathers = []
            for s in range(nb):
                j = j8 * nb + s
                gathers.append(pltpu.async_copy(
                    tf_hbm.at[idx_v.at[j]], fbuf.at[s], sem_g))
                gathers.append(pltpu.async_copy(
                    tp_hbm.at[idx_v.at[j]], pbuf.at[s], sem_g))
            for g in gathers:
                g.wait()
            stores = []
            for s in range(nb):
                base = wid * per_w + (j8 * nb + s) * lane
                stores.append(pltpu.async_copy(
                    fbuf.at[s], gf_hbm.at[pl.ds(base, lane)], sem_o))
                stores.append(pltpu.async_copy(
                    pbuf.at[s], gp_hbm.at[pl.ds(base, lane)], sem_o))
            for st in stores:
                st.wait()
            return carry

        lax.fori_loop(0, nch // nb, body, 0)

    return run(tf, tp, idx3)


def _attn_body(r4, cnt, feats_ref, pts_ref, gf_ref, gp_ref,
               wq_ref, wk_ref, wv_ref, wg1_ref, wg2_ref, wd1_ref, wd2_ref,
               bq_ref, bk_ref, bv_ref, bg1_ref, bg2_ref, bd1_ref, bd2_ref,
               w1_ref, b1_ref, w2_ref, b2_ref, w3_ref, b3_ref,
               out_ref, acc, stats):
    p = pl.program_id(0)
    i = pl.program_id(1)
    rk = _K * r4

    def t_k(x):
        return jnp.concatenate([x] * _K, axis=0)

    def s_k(x):
        a = x[0:r4]
        for k in range(1, _K):
            a = a + x[k * r4:(k + 1) * r4]
        return a

    def m_k(x):
        a = x[0:r4]
        for k in range(1, _K):
            a = jnp.maximum(a, x[k * r4:(k + 1) * r4])
        return a

    @pl.when(jnp.logical_and(p == 0, i == 0))
    def _init():
        acc[...] = jnp.zeros_like(acc)

    def _fold4(v, l):
        return v[:, 0:l] + v[:, l:2 * l] + v[:, 2 * l:3 * l] + v[:, 3 * l:4 * l]

    def _finalize(src, dst, l):
        s = _fold4(acc[src:src + 1, 0:4 * l], l) * (1.0 / cnt)
        ss = _fold4(acc[src + 1:src + 2, 0:4 * l], l) * (1.0 / cnt)
        r = lax.rsqrt(ss - s * s + _EPS)
        stats[dst:dst + 1, 0:4 * l] = jnp.concatenate([s] * 4, axis=1)
        stats[dst + 1:dst + 2, 0:4 * l] = jnp.concatenate([r] * 4, axis=1)

    @pl.when(jnp.logical_and(p == 1, i == 0))
    def _fin1():
        _finalize(0, 0, 16)

    @pl.when(jnp.logical_and(p == 2, i == 0))
    def _fin2():
        _finalize(2, 2, 32)

    @pl.when(jnp.logical_and(p == 3, i == 0))
    def _fin3():
        _finalize(4, 4, 32)

    gp2 = gp_ref[...].reshape(rk, 64)
    dp = t_k(pts_ref[...]) - gp2
    pos_raw = _dot(dp, wd1_ref[...]) + bd1_ref[...]

    @pl.when(p == 0)
    def _acc1():
        acc[0:1, 0:64] += jnp.sum(pos_raw, axis=0, keepdims=True)
        acc[1:2, 0:64] += jnp.sum(pos_raw * pos_raw, axis=0, keepdims=True)

    @pl.when(p >= 1)
    def _main():
        pos_d = _gelu((pos_raw - stats[0:1, 0:64]) * stats[1:2, 0:64]
                      * w1_ref[...] + b1_ref[...])
        pos32 = _dot(pos_d, wd2_ref[...]) + bd2_ref[...]
        q = _dot(feats_ref[...], wq_ref[...]) + bq_ref[...]
        gf2 = gf_ref[...].reshape(rk, 128)
        kk = _dot(gf2, wk_ref[...]) + bk_ref[...]
        gamma0 = t_k(q) - kk + pos32

        @pl.when(p == 1)
        def _acc2():
            acc[2:3, :] += jnp.sum(gamma0, axis=0, keepdims=True)
            acc[3:4, :] += jnp.sum(gamma0 * gamma0, axis=0, keepdims=True)

        @pl.when(p >= 2)
        def _main2():
            g1 = _gelu((gamma0 - stats[2:3, :]) * stats[3:4, :]
                       * w2_ref[...] + b2_ref[...])
            g2 = _dot(g1, wg1_ref[...]) + bg1_ref[...]

            @pl.when(p == 2)
            def _acc3():
                acc[4:5, :] += jnp.sum(g2, axis=0, keepdims=True)
                acc[5:6, :] += jnp.sum(g2 * g2, axis=0, keepdims=True)

            @pl.when(p == 3)
            def _final():
                g3 = _gelu((g2 - stats[4:5, :]) * stats[5:6, :]
                           * w3_ref[...] + b3_ref[...])
                g4 = _dot(g3, wg2_ref[...]) + bg2_ref[...]
                mx = m_k(g4)
                e = jnp.exp(g4 - t_k(mx))
                rho = e / t_k(s_k(e))
                val = _dot(gf2, wv_ref[...]) + bv_ref[...] + pos32
                out_ref[...] = s_k(rho * val)


def _attn(featsp, ptsp, gfp, gpp, wmats, wvecs, r4=128):
    bn4 = featsp.shape[0]
    nblk = bn4 // r4
    cnt = float(bn4 * 4 * _K)
    full = lambda s: pl.BlockSpec(s, lambda p, i: (0,) * len(s))
    in_specs = (
        [pl.BlockSpec((r4, 128), lambda p, i: (i, 0)),
         pl.BlockSpec((r4, 64), lambda p, i: (i, 0)),
         pl.BlockSpec((_K, r4, 128), lambda p, i: (0, i, 0)),
         pl.BlockSpec((_K, r4, 64), lambda p, i: (0, i, 0))]
        + [full(w.shape) for w in wmats]
        + [full(v.shape) for v in wvecs]
    )
    return pl.pallas_call(
        functools.partial(_attn_body, r4, cnt),
        grid=(4, nblk),
        in_specs=in_specs,
        out_specs=pl.BlockSpec((r4, 128), lambda p, i: (i, 0)),
        out_shape=jax.ShapeDtypeStruct((bn4, 128), jnp.float32),
        scratch_shapes=[pltpu.VMEM((8, 128), jnp.float32),
                        pltpu.VMEM((8, 128), jnp.float32)],
    )(featsp, ptsp, gfp, gpp, *wmats, *wvecs)


def kernel(feats, points, params):
    bb, n, df = feats.shape
    bn = bb * n
    f32 = jnp.float32

    tp = jnp.pad(points.reshape(bn, 3), ((0, 0), (0, 13)))
    gfs, gps = [], []
    for b in range(bb):
        idx_b = _knn(points[b:b + 1])
        idx3 = idx_b.reshape(32, (_K * n) // (32 * 128), 128)
        gf_b, gp_b = _gather(feats[b], tp[b * n:(b + 1) * n], idx3)
        gfs.append(gf_b.reshape(_K, n // 4, 128))
        gps.append(gp_b.reshape(_K, n // 4, 64))
    gf = jnp.stack(gfs, axis=1)
    gp = jnp.stack(gps, axis=1)

    eye4 = jnp.eye(4, dtype=f32)
    kron4 = lambda w: jnp.kron(eye4, w)
    wd1_16 = jnp.zeros((16, 16), f32).at[0:3, 0:3].set(params['Wd1'])
    wd2_16 = jnp.zeros((16, df), f32).at[0:3, :].set(params['Wd2'])
    wmats = [kron4(params['Wq']), kron4(params['Wk']), kron4(params['Wv']),
             kron4(params['Wg1']), kron4(params['Wg2']),
             kron4(wd1_16), kron4(wd2_16)]
    t4 = lambda v: jnp.tile(v, 4).reshape(1, -1)
    pad16 = lambda v: jnp.pad(v, (0, 13))
    wvecs = [t4(params['bq']), t4(params['bk']), t4(params['bv']),
             t4(params['bg1']), t4(params['bg2']),
             t4(pad16(params['bd1'])), t4(params['bd2']),
             t4(pad16(params['bn_delta_w'])), t4(pad16(params['bn_delta_b'])),
             t4(params['bn_g1_w']), t4(params['bn_g1_b']),
             t4(params['bn_g2_w']), t4(params['bn_g2_b'])]

    featsp = feats.reshape(bn // 4, 128)
    ptsp = tp.reshape(bn // 4, 64)
    gfp = gf.reshape(_K, bn // 4, 128)
    gpp = gp.reshape(_K, bn // 4, 64)
    outp = _attn(featsp, ptsp, gfp, gpp, wmats, wvecs)
    return outp.reshape(bb, n, df)
